# baseline (speedup 1.0000x reference)
import functools

import jax
import jax.numpy as jnp
from jax import lax
from jax.experimental import pallas as pl
from jax.experimental.pallas import tpu as pltpu
from jax.experimental.pallas import tpu_sc as plsc

DIM = 64
SCALE = 8.0
NC, NS, LANES = 2, 16, 16
NW = NC * NS
NBUF = 4
TR = 2
IDXSEG = 80


def kernel(tokens, table):
    B, L = tokens.shape
    rows_w = B // NW
    per_w = rows_w * L
    nch = rows_w // TR
    nt = nch // NBUF
    CH = TR * L
    nseg = CH // IDXSEG

    idx = tokens.astype(jnp.int32).reshape(NW, per_w)

    mesh = plsc.VectorSubcoreMesh(core_axis_name="c", subcore_axis_name="s")

    @functools.partial(
        pl.kernel,
        mesh=mesh,
        compiler_params=pltpu.CompilerParams(use_tc_tiling_on_sc=False),
        out_type=jax.ShapeDtypeStruct((B, L, DIM), jnp.float32),
        scratch_types=[
            pltpu.VMEM((per_w,), jnp.int32),
            pltpu.VMEM((NBUF, CH, DIM), jnp.float32),
        ]
        + [pltpu.SemaphoreType.DMA] * (2 * NBUF),
    )
    def emb_kernel(tok_hbm, tab_hbm, out_hbm, idx_v, rows_v, *sems):
        gsem = sems[:NBUF]
        osem = sems[NBUF:]
        wid = lax.axis_index("s") * NC + lax.axis_index("c")
        base = wid * rows_w
        pltpu.sync_copy(tok_hbm.at[wid], idx_v)

        def issue_gathers(c, b):
            for s in range(nseg):
                pltpu.async_copy(
                    tab_hbm.at[idx_v.at[pl.ds(c * CH + s * IDXSEG, IDXSEG)]],
                    rows_v.at[b].at[pl.ds(s * IDXSEG, IDXSEG)],
                    gsem[b],
                )

        def drain_gathers(c, b):
            for s in range(nseg):
                pltpu.make_async_copy(
                    tab_hbm.at[idx_v.at[pl.ds(c * CH + s * IDXSEG, IDXSEG)]],
                    rows_v.at[b].at[pl.ds(s * IDXSEG, IDXSEG)],
                    gsem[b],
                ).wait()

        def store_row(c, b, j):
            return pltpu.make_async_copy(
                rows_v.at[b].at[pl.ds(j * L, L)],
                out_hbm.at[base + c * TR + j],
                osem[b],
            )

        def store_start(c, b):
            for j in range(TR):
                store_row(c, b, j).start()

        def store_wait(c, b):
            for j in range(TR):
                store_row(c, b, j).wait()

        issue_gathers(0, 0)
        issue_gathers(1, 1)

        @pl.loop(0, nt)
        def outer(t):
            for b in range(NBUF):
                c = t * NBUF + b
                nb = (b + 2) % NBUF
                if b < 2:
                    @pl.when(t > 0)
                    def _():
                        store_wait(c - 2, nb)

                    issue_gathers(c + 2, nb)
                else:
                    @pl.when(t < nt - 1)
                    def _():
                        store_wait(c - 2, nb)
                        issue_gathers(c + 2, nb)

                drain_gathers(c, b)

                @plsc.parallel_loop(0, CH, 1, unroll=8)
                def scale_loop(r):
                    for q in range(DIM // LANES):
                        sl = pl.ds(q * LANES, LANES)
                        rows_v[b, r, sl] = rows_v[b, r, sl] * SCALE

                store_start(c, b)

        for b in range(NBUF):
            store_wait(nch - NBUF + b, b)

    return emb_kernel(idx, table)
